# baseline (speedup 1.0000x reference)
import jax
import jax.numpy as jnp
from jax import lax
from jax.experimental import pallas as pl
from jax.experimental.pallas import tpu as pltpu
from jax.experimental.pallas import tpu_sc as plsc

N = 10000
E = 320000
D_IN = 128
HIDDEN = 64
HEADS = 4
N_CLASSES = 16
ORDER = 16

NT = 16
N_PAD = 10240
RPT = N_PAD // NT
EPT = E // NT
CHUNK = 128
BCH = 8
NBLK = 20
EPT_PAD = NBLK * BCH * CHUNK
PAD_ROW = N_PAD - 1
NRC = 128
NROWCH = RPT // NRC


def _rsqrt_sc(v):
    vi = plsc.bitcast(v, jnp.int32)
    yi = jnp.full((16,), 0x5F3759DF, jnp.int32) - (vi >> 1)
    y = plsc.bitcast(yi, jnp.float32)
    for _ in range(4):
        y = y * (1.5 - 0.5 * v * y * y)
    return y


def _rep16(ref, r):
    return plsc.load_gather(ref, [jnp.full((16,), r, jnp.int32)])


def _make_filter(nsplit, fh):
    mesh = plsc.VectorSubcoreMesh(core_axis_name="c", subcore_axis_name="s")
    nq = fh // 16

    def body(h, srcs, dsts, u_all, srec, sb0, sb1, db0, db1, g0,
             bones, bzd, bdeg, bdis, bdis2, sh_acc, sh_deg,
             semi0, semi1, semg0, semg1, sems0, sems1):
        c = lax.axis_index("c")
        s = lax.axis_index("s")
        base = s * RPT

        def gated(fn):
            if nsplit == 1:
                pl.when(c == 0)(fn)
            else:
                fn()
        co = c if nsplit == 2 else 0

        def _p0():
            def wz(r, _):
                for q in range(nq):
                    g0[r, pl.ds(16 * q, 16)] = jnp.zeros((16,), jnp.float32)
                return 0
            lax.fori_loop(0, 2 * NRC, wz, 0)

            def wo(m, _):
                bones[pl.ds(16 * m, 16)] = jnp.ones((16,), jnp.float32)
                return 0
            lax.fori_loop(0, CHUNK // 16, wo, 0)

            def wd(m, _):
                bzd[pl.ds(16 * m, 16)] = jnp.zeros((16,), jnp.float32)
                return 0
            lax.fori_loop(0, RPT // 16, wd, 0)

            def zch(j, _):
                pltpu.sync_copy(g0.at[pl.ds(0, NRC)],
                                sh_acc.at[pl.ds(base + j * NRC, NRC)])
                return 0
            lax.fori_loop(0, NROWCH, zch, 0)
            pltpu.sync_copy(bzd, sh_deg.at[pl.ds(base, RPT)])
        gated(_p0)
        plsc.subcore_barrier()

        def _p1():
            def dblk(b, _):
                pltpu.sync_copy(dsts.at[s, b], db0)
                for j in range(BCH):
                    pltpu.sync_copy(bones, sh_deg.at[db0.at[j]], add=True)
                return 0
            lax.fori_loop(0, NBLK, dblk, 0)
        gated(_p1)
        plsc.subcore_barrier()

        def _p2():
            pltpu.sync_copy(sh_deg.at[pl.ds(base, RPT)], bdeg)

            def prow(m, _):
                dv = bdeg[pl.ds(16 * m, 16)]
                pos = dv > 0.0
                y = jnp.where(pos, _rsqrt_sc(jnp.maximum(dv, 1.0)), 0.0)
                d2 = jnp.where(pos, 1.0 / jnp.maximum(dv, 1.0), 0.0)
                bdis[pl.ds(16 * m, 16)] = y
                bdis2[pl.ds(16 * m, 16)] = d2
                bdeg[pl.ds(16 * m, 16)] = dv * y
                return 0
            lax.fori_loop(0, RPT // 16, prow, 0)

            @pl.when(c == 0)
            def _():
                pltpu.sync_copy(bdeg, srec.at[pl.ds(base, RPT)])
        gated(_p2)

        def _p3():
            def uch(j, _):
                r0 = base + j * NRC
                pltpu.sync_copy(h.at[co, pl.ds(r0, NRC)], g0.at[pl.ds(0, NRC)])

                def urow(r, _):
                    dv = _rep16(bdis, j * NRC + r)
                    for q in range(nq):
                        g0[r, pl.ds(16 * q, 16)] = dv * g0[r, pl.ds(16 * q, 16)]
                    return 0
                lax.fori_loop(0, NRC, urow, 0)
                pltpu.sync_copy(g0.at[pl.ds(0, NRC)],
                                u_all.at[0, co, pl.ds(r0, NRC)])
                return 0
            lax.fori_loop(0, NROWCH, uch, 0)
        gated(_p3)
        plsc.subcore_barrier()

        def step(i, _):
            usrc = u_all.at[i, co]

            def _gstart(sb, j, buf, sem):
                pltpu.async_copy(usrc.at[sb.at[j]], buf, sem)

            def _gwait(sb, j, buf, sem):
                pltpu.make_async_copy(usrc.at[sb.at[j]], buf, sem).wait()

            def _sstart(db, j, buf, sem):
                pltpu.async_copy(buf, sh_acc.at[db.at[j]], sem, add=True)

            def _swait(db, j, buf, sem):
                pltpu.make_async_copy(buf, sh_acc.at[db.at[j]], sem).wait()

            def _blk(b, sb, db, semi, nxt):
                pltpu.make_async_copy(srcs.at[s, b], sb, semi).wait()
                pltpu.make_async_copy(dsts.at[s, b], db, semi).wait()
                if nxt is not None:
                    bn, sbn, dbn, semn = nxt
                    pltpu.async_copy(srcs.at[s, bn], sbn, semn)
                    pltpu.async_copy(dsts.at[s, bn], dbn, semn)
                bufs = (g0.at[pl.ds(0, NRC)], g0.at[pl.ds(NRC, NRC)])
                semgs = (semg0, semg1)
                semss = (sems0, sems1)
                _gstart(sb, 0, bufs[0], semg0)
                for j in range(BCH):
                    p, o = j % 2, (j + 1) % 2
                    if j + 1 < BCH:
                        if j >= 1:
                            _swait(db, j - 1, bufs[o], semss[o])
                        _gstart(sb, j + 1, bufs[o], semgs[o])
                    _gwait(sb, j, bufs[p], semgs[p])
                    _sstart(db, j, bufs[p], semss[p])
                _swait(db, BCH - 2, bufs[BCH % 2], semss[BCH % 2])
                _swait(db, BCH - 1, bufs[(BCH + 1) % 2], semss[(BCH + 1) % 2])

            def _scat():
                pltpu.async_copy(srcs.at[s, 0], sb0, semi0)
                pltpu.async_copy(dsts.at[s, 0], db0, semi0)

                def pairs(q, _):
                    b0 = 2 * q
                    _blk(b0, sb0, db0, semi0, (b0 + 1, sb1, db1, semi1))

                    @pl.when(q < NBLK // 2 - 1)
                    def _():
                        pltpu.async_copy(srcs.at[s, b0 + 2], sb0, semi0)
                        pltpu.async_copy(dsts.at[s, b0 + 2], db0, semi0)
                    _blk(b0 + 1, sb1, db1, semi1, None)
                    return 0
                lax.fori_loop(0, NBLK // 2, pairs, 0)
            gated(_scat)
            plsc.subcore_barrier()

            fa = jnp.where(i == 0, -1.0, -2.0)
            fb = jnp.where(i == 0, 0.0, -1.0)
            im1 = lax.max(i - 1, 0)

            def _node():
                def nch(j, _):
                    r0 = base + j * NRC
                    pltpu.sync_copy(sh_acc.at[pl.ds(r0, NRC)],
                                    g0.at[pl.ds(0, NRC)])
                    pltpu.sync_copy(u_all.at[im1, co, pl.ds(r0, NRC)],
                                    g0.at[pl.ds(NRC, NRC)])

                    def nrow(r, _):
                        d2 = _rep16(bdis2, j * NRC + r)
                        for q in range(nq):
                            sl = pl.ds(16 * q, 16)
                            g0[r, sl] = (fa * d2 * g0[r, sl]
                                         + fb * g0[NRC + r, sl])
                            g0[NRC + r, sl] = jnp.zeros((16,), jnp.float32)
                        return 0
                    lax.fori_loop(0, NRC, nrow, 0)
                    pltpu.sync_copy(g0.at[pl.ds(0, NRC)],
                                    u_all.at[i + 1, co, pl.ds(r0, NRC)])
                    pltpu.sync_copy(g0.at[pl.ds(NRC, NRC)],
                                    sh_acc.at[pl.ds(r0, NRC)])
                    return 0
                lax.fori_loop(0, NROWCH, nch, 0)
            gated(_node)
            plsc.subcore_barrier()
            return 0
        lax.fori_loop(0, ORDER, step, 0)

    return pl.kernel(
        body,
        out_type=(
            jax.ShapeDtypeStruct((ORDER + 1, nsplit, N_PAD, fh), jnp.float32),
            jax.ShapeDtypeStruct((N_PAD,), jnp.float32),
        ),
        mesh=mesh,
        compiler_params=pltpu.CompilerParams(needs_layout_passes=False),
        scratch_types=[
            pltpu.VMEM((BCH, CHUNK), jnp.int32),
            pltpu.VMEM((BCH, CHUNK), jnp.int32),
            pltpu.VMEM((BCH, CHUNK), jnp.int32),
            pltpu.VMEM((BCH, CHUNK), jnp.int32),
            pltpu.VMEM((2 * NRC, fh), jnp.float32),
            pltpu.VMEM((CHUNK,), jnp.float32),
            pltpu.VMEM((RPT,), jnp.float32),
            pltpu.VMEM((RPT,), jnp.float32),
            pltpu.VMEM((RPT,), jnp.float32),
            pltpu.VMEM((RPT,), jnp.float32),
            pltpu.VMEM_SHARED((N_PAD, fh), jnp.float32),
            pltpu.VMEM_SHARED((N_PAD,), jnp.float32),
            pltpu.SemaphoreType.DMA,
            pltpu.SemaphoreType.DMA,
            pltpu.SemaphoreType.DMA,
            pltpu.SemaphoreType.DMA,
            pltpu.SemaphoreType.DMA,
            pltpu.SemaphoreType.DMA,
        ],
        name=f"cheb_filter_sc_f{fh}x{nsplit}",
    )


_BR = 256


def _tc1_body(x_ref, w_ref, o_ref):
    o_ref[...] = jnp.dot(x_ref[...], w_ref[...],
                         preferred_element_type=jnp.float32)[None]


def _tc1(xp, W1):
    return pl.pallas_call(
        _tc1_body,
        grid=(N_PAD // _BR, 2),
        in_specs=[
            pl.BlockSpec((_BR, D_IN), lambda i, c: (i, 0)),
            pl.BlockSpec((D_IN, 128), lambda i, c: (0, c)),
        ],
        out_specs=pl.BlockSpec((1, _BR, 128), lambda i, c: (c, i, 0)),
        out_shape=jax.ShapeDtypeStruct((2, N_PAD, 128), jnp.float32),
    )(xp, W1)


def _tc2_body(coef_ref, ciso_ref, u_ref, h_ref, srec_ref, w2_ref,
              l1_ref, h2_ref):
    sr = srec_ref[:, 0:1]
    halves = []
    for c in (0, 1):
        hh = h_ref[c]
        outs = jnp.zeros_like(hh)
        for k in range(1, ORDER + 1):
            row = jnp.concatenate(
                [jnp.full((1, 64), coef_ref[2 * c, k], jnp.float32),
                 jnp.full((1, 64), coef_ref[2 * c + 1, k], jnp.float32)],
            axis=1)
            outs = outs + row * u_ref[k, c]
        c0row = jnp.concatenate(
            [jnp.full((1, 64), coef_ref[2 * c, 0], jnp.float32),
             jnp.full((1, 64), coef_ref[2 * c + 1, 0], jnp.float32)], axis=1)
        cirow = jnp.concatenate(
            [jnp.full((1, 64), ciso_ref[2 * c], jnp.float32),
             jnp.full((1, 64), ciso_ref[2 * c + 1], jnp.float32)], axis=1)
        t = jnp.where(sr > 0.0, c0row * hh + sr * outs, cirow * hh)
        halves.append(jnp.where(t > 0.0, t, jnp.exp(t) - 1.0))
    l1 = jnp.concatenate(halves, axis=1)
    l1_ref[...] = l1
    h2_ref[...] = jnp.dot(l1, w2_ref[...], preferred_element_type=jnp.float32)


def _tc2(u1, hh, srec, W2, coef1, ciso1):
    return pl.pallas_call(
        _tc2_body,
        grid=(N_PAD // _BR,),
        in_specs=[
            pl.BlockSpec(memory_space=pltpu.SMEM),
            pl.BlockSpec(memory_space=pltpu.SMEM),
            pl.BlockSpec((ORDER + 1, 2, _BR, 128), lambda i: (0, 0, i, 0)),
            pl.BlockSpec((2, _BR, 128), lambda i: (0, i, 0)),
            pl.BlockSpec((_BR, 1), lambda i: (i, 0)),
            pl.BlockSpec((HEADS * HIDDEN, N_CLASSES), lambda i: (0, 0)),
        ],
        out_specs=[
            pl.BlockSpec((_BR, HEADS * HIDDEN), lambda i: (i, 0)),
            pl.BlockSpec((_BR, N_CLASSES), lambda i: (i, 0)),
        ],
        out_shape=[
            jax.ShapeDtypeStruct((N_PAD, HEADS * HIDDEN), jnp.float32),
            jax.ShapeDtypeStruct((N_PAD, N_CLASSES), jnp.float32),
        ],
    )(coef1, ciso1, u1, hh, srec, W2)


def _tc3_body(coef_ref, ciso_ref, u_ref, h2_ref, srec_ref, lp_ref, l2_ref):
    sr = srec_ref[:, 0:1]
    h2 = h2_ref[...]
    outs = jnp.zeros_like(h2)
    for k in range(1, ORDER + 1):
        outs = outs + coef_ref[0, k] * u_ref[k, 0, :, :N_CLASSES]
    l2 = jnp.where(sr > 0.0, coef_ref[0, 0] * h2 + sr * outs,
                   ciso_ref[0] * h2)
    m = jnp.max(l2, axis=1, keepdims=True)
    sh = l2 - m
    lse = jnp.log(jnp.sum(jnp.exp(sh), axis=1, keepdims=True))
    l2_ref[...] = l2
    lp_ref[...] = sh - lse


def _tc3(u2, h2, srec, coef2, ciso2):
    return pl.pallas_call(
        _tc3_body,
        grid=(N_PAD // _BR,),
        in_specs=[
            pl.BlockSpec(memory_space=pltpu.SMEM),
            pl.BlockSpec(memory_space=pltpu.SMEM),
            pl.BlockSpec((ORDER + 1, 1, _BR, 128), lambda i: (0, 0, i, 0)),
            pl.BlockSpec((_BR, 16), lambda i: (i, 0)),
            pl.BlockSpec((_BR, 1), lambda i: (i, 0)),
        ],
        out_specs=[
            pl.BlockSpec((_BR, 16), lambda i: (i, 0)),
            pl.BlockSpec((_BR, 16), lambda i: (i, 0)),
        ],
        out_shape=[
            jax.ShapeDtypeStruct((N_PAD, N_CLASSES), jnp.float32),
            jax.ShapeDtypeStruct((N_PAD, N_CLASSES), jnp.float32),
        ],
    )(coef2, ciso2, u2, h2, srec)


def kernel(x, edge_index, W1, coef1, W2, coef2):
    src = edge_index[0].astype(jnp.int32)
    dst = edge_index[1].astype(jnp.int32)
    src, dst = lax.sort_key_val(src, dst)
    pad_e = EPT_PAD - EPT
    s4 = jnp.pad(src.reshape(NT, EPT), ((0, 0), (0, pad_e)),
                 constant_values=PAD_ROW).reshape(NT, NBLK, BCH, CHUNK)
    d4 = jnp.pad(dst.reshape(NT, EPT), ((0, 0), (0, pad_e)),
                 constant_values=PAD_ROW).reshape(NT, NBLK, BCH, CHUNK)
    xp = jnp.pad(x, ((0, N_PAD - N), (0, 0)))

    alt = ((-1.0) ** jnp.arange(ORDER // 2 + 1)).astype(jnp.float32)
    ciso1 = (coef1[:, ::2] * alt[None, :]).sum(axis=1)
    ciso2 = (coef2[:, ::2] * alt[None, :]).sum(axis=1)

    hh = _tc1(xp, W1)
    u1, srec = _make_filter(2, 128)(hh, s4, d4)
    srec2 = srec[:, None]
    l1p, h2p = _tc2(u1, hh, srec2, W2, coef1, ciso1)
    h2w = jnp.pad(h2p, ((0, 0), (0, 128 - N_CLASSES)))
    u2, _srec_b = _make_filter(1, 128)(h2w[None], s4, d4)
    lpp, l2p = _tc3(u2, h2p, srec2, coef2, ciso2)
    return lpp[:N], l2p[:N], l1p[:N]


# speedup vs baseline: 1.3676x; 1.3676x over previous
import jax
import jax.numpy as jnp
from jax import lax
from jax.experimental import pallas as pl
from jax.experimental.pallas import tpu as pltpu
from jax.experimental.pallas import tpu_sc as plsc

N = 10000
E = 320000
D_IN = 128
HIDDEN = 64
HEADS = 4
N_CLASSES = 16
ORDER = 16

NT = 16
N_PAD = 10240
RPT = N_PAD // NT
EPT = E // NT
CHUNK = 128
BCH = 16
NBLK = 10
EPT_PAD = NBLK * BCH * CHUNK
PAD_ROW = N_PAD - 1
NRC = 128
NROWCH = RPT // NRC


def _rsqrt_sc(v):
    vi = plsc.bitcast(v, jnp.int32)
    yi = jnp.full((16,), 0x5F3759DF, jnp.int32) - (vi >> 1)
    y = plsc.bitcast(yi, jnp.float32)
    for _ in range(4):
        y = y * (1.5 - 0.5 * v * y * y)
    return y


def _rep16(ref, r):
    return plsc.load_gather(ref, [jnp.full((16,), r, jnp.int32)])


def _make_filter(nsplit, fh):
    mesh = plsc.VectorSubcoreMesh(core_axis_name="c", subcore_axis_name="s")
    nq = fh // 16

    def body(h, srcs, dsts, u_all, srec, sb0, sb1, db0, db1, g0,
             bones, bzd, bdeg, bdis, bdis2, sh_acc, sh_deg,
             semi0, semi1, semg0, semg1, sems0, sems1):
        c = lax.axis_index("c")
        s = lax.axis_index("s")
        base = s * RPT

        def gated(fn):
            if nsplit == 1:
                pl.when(c == 0)(fn)
            else:
                fn()
        co = c if nsplit == 2 else 0

        def _p0():
            def wz(r, _):
                for q in range(nq):
                    g0[r, pl.ds(16 * q, 16)] = jnp.zeros((16,), jnp.float32)
                return 0
            lax.fori_loop(0, 2 * NRC, wz, 0)

            def wo(m, _):
                bones[pl.ds(16 * m, 16)] = jnp.ones((16,), jnp.float32)
                return 0
            lax.fori_loop(0, CHUNK // 16, wo, 0)

            def wd(m, _):
                bzd[pl.ds(16 * m, 16)] = jnp.zeros((16,), jnp.float32)
                return 0
            lax.fori_loop(0, RPT // 16, wd, 0)

            def zch(j, _):
                pltpu.sync_copy(g0.at[pl.ds(0, NRC)],
                                sh_acc.at[pl.ds(base + j * NRC, NRC)])
                return 0
            lax.fori_loop(0, NROWCH, zch, 0)
            pltpu.sync_copy(bzd, sh_deg.at[pl.ds(base, RPT)])
        gated(_p0)
        plsc.subcore_barrier()

        def _p1():
            def dblk(b, _):
                pltpu.sync_copy(dsts.at[s, b], db0)
                for j in range(BCH):
                    pltpu.sync_copy(bones, sh_deg.at[db0.at[j]], add=True)
                return 0
            lax.fori_loop(0, NBLK, dblk, 0)
        gated(_p1)
        plsc.subcore_barrier()

        def _p2():
            pltpu.sync_copy(sh_deg.at[pl.ds(base, RPT)], bdeg)

            def prow(m, _):
                dv = bdeg[pl.ds(16 * m, 16)]
                pos = dv > 0.0
                y = jnp.where(pos, _rsqrt_sc(jnp.maximum(dv, 1.0)), 0.0)
                d2 = jnp.where(pos, 1.0 / jnp.maximum(dv, 1.0), 0.0)
                bdis[pl.ds(16 * m, 16)] = y
                bdis2[pl.ds(16 * m, 16)] = d2
                bdeg[pl.ds(16 * m, 16)] = dv * y
                return 0
            lax.fori_loop(0, RPT // 16, prow, 0)

            @pl.when(c == 0)
            def _():
                pltpu.sync_copy(bdeg, srec.at[pl.ds(base, RPT)])
        gated(_p2)

        def _p3():
            def uch(j, _):
                r0 = base + j * NRC
                pltpu.sync_copy(h.at[co, pl.ds(r0, NRC)], g0.at[pl.ds(0, NRC)])

                def urow(r, _):
                    dv = _rep16(bdis, j * NRC + r)
                    for q in range(nq):
                        g0[r, pl.ds(16 * q, 16)] = dv * g0[r, pl.ds(16 * q, 16)]
                    return 0
                lax.fori_loop(0, NRC, urow, 0)
                pltpu.sync_copy(g0.at[pl.ds(0, NRC)],
                                u_all.at[0, co, pl.ds(r0, NRC)])
                return 0
            lax.fori_loop(0, NROWCH, uch, 0)
        gated(_p3)
        plsc.subcore_barrier()

        def step(i, _):
            usrc = u_all.at[i, co]

            def _gstart(sb, j, buf, sem):
                pltpu.async_copy(usrc.at[sb.at[j]], buf, sem)

            def _gwait(sb, j, buf, sem):
                pltpu.make_async_copy(usrc.at[sb.at[j]], buf, sem).wait()

            def _sstart(db, j, buf, sem):
                pltpu.async_copy(buf, sh_acc.at[db.at[j]], sem, add=True)

            def _swait(db, j, buf, sem):
                pltpu.make_async_copy(buf, sh_acc.at[db.at[j]], sem).wait()

            def _blk(b, sb, db, semi, nxt):
                pltpu.make_async_copy(srcs.at[s, b], sb, semi).wait()
                pltpu.make_async_copy(dsts.at[s, b], db, semi).wait()
                if nxt is not None:
                    bn, sbn, dbn, semn = nxt
                    pltpu.async_copy(srcs.at[s, bn], sbn, semn)
                    pltpu.async_copy(dsts.at[s, bn], dbn, semn)
                bufs = (g0.at[pl.ds(0, NRC)], g0.at[pl.ds(NRC, NRC)])
                semgs = (semg0, semg1)
                semss = (sems0, sems1)
                _gstart(sb, 0, bufs[0], semg0)
                for j in range(BCH):
                    p, o = j % 2, (j + 1) % 2
                    if j + 1 < BCH:
                        if j >= 1:
                            _swait(db, j - 1, bufs[o], semss[o])
                        _gstart(sb, j + 1, bufs[o], semgs[o])
                    _gwait(sb, j, bufs[p], semgs[p])
                    _sstart(db, j, bufs[p], semss[p])
                _swait(db, BCH - 2, bufs[BCH % 2], semss[BCH % 2])
                _swait(db, BCH - 1, bufs[(BCH + 1) % 2], semss[(BCH + 1) % 2])

            def _scat():
                pltpu.async_copy(srcs.at[s, 0], sb0, semi0)
                pltpu.async_copy(dsts.at[s, 0], db0, semi0)

                def pairs(q, _):
                    b0 = 2 * q
                    _blk(b0, sb0, db0, semi0, (b0 + 1, sb1, db1, semi1))

                    @pl.when(q < NBLK // 2 - 1)
                    def _():
                        pltpu.async_copy(srcs.at[s, b0 + 2], sb0, semi0)
                        pltpu.async_copy(dsts.at[s, b0 + 2], db0, semi0)
                    _blk(b0 + 1, sb1, db1, semi1, None)
                    return 0
                lax.fori_loop(0, NBLK // 2, pairs, 0)
            gated(_scat)
            plsc.subcore_barrier()

            fa = jnp.where(i == 0, -1.0, -2.0)
            fb = jnp.where(i == 0, 0.0, -1.0)
            im1 = lax.max(i - 1, 0)

            def _node():
                def nch(j, _):
                    r0 = base + j * NRC
                    pltpu.sync_copy(sh_acc.at[pl.ds(r0, NRC)],
                                    g0.at[pl.ds(0, NRC)])
                    pltpu.sync_copy(u_all.at[im1, co, pl.ds(r0, NRC)],
                                    g0.at[pl.ds(NRC, NRC)])

                    def nrow(r, _):
                        d2 = _rep16(bdis2, j * NRC + r)
                        for q in range(nq):
                            sl = pl.ds(16 * q, 16)
                            g0[r, sl] = (fa * d2 * g0[r, sl]
                                         + fb * g0[NRC + r, sl])
                            g0[NRC + r, sl] = jnp.zeros((16,), jnp.float32)
                        return 0
                    lax.fori_loop(0, NRC, nrow, 0)
                    pltpu.sync_copy(g0.at[pl.ds(0, NRC)],
                                    u_all.at[i + 1, co, pl.ds(r0, NRC)])
                    pltpu.sync_copy(g0.at[pl.ds(NRC, NRC)],
                                    sh_acc.at[pl.ds(r0, NRC)])
                    return 0
                lax.fori_loop(0, NROWCH, nch, 0)
            gated(_node)
            plsc.subcore_barrier()
            return 0
        lax.fori_loop(0, ORDER, step, 0)

    return pl.kernel(
        body,
        out_type=(
            jax.ShapeDtypeStruct((ORDER + 1, nsplit, N_PAD, fh), jnp.float32),
            jax.ShapeDtypeStruct((N_PAD,), jnp.float32),
        ),
        mesh=mesh,
        compiler_params=pltpu.CompilerParams(needs_layout_passes=False),
        scratch_types=[
            pltpu.VMEM((BCH, CHUNK), jnp.int32),
            pltpu.VMEM((BCH, CHUNK), jnp.int32),
            pltpu.VMEM((BCH, CHUNK), jnp.int32),
            pltpu.VMEM((BCH, CHUNK), jnp.int32),
            pltpu.VMEM((2 * NRC, fh), jnp.float32),
            pltpu.VMEM((CHUNK,), jnp.float32),
            pltpu.VMEM((RPT,), jnp.float32),
            pltpu.VMEM((RPT,), jnp.float32),
            pltpu.VMEM((RPT,), jnp.float32),
            pltpu.VMEM((RPT,), jnp.float32),
            pltpu.VMEM_SHARED((N_PAD, fh), jnp.float32),
            pltpu.VMEM_SHARED((N_PAD,), jnp.float32),
            pltpu.SemaphoreType.DMA,
            pltpu.SemaphoreType.DMA,
            pltpu.SemaphoreType.DMA,
            pltpu.SemaphoreType.DMA,
            pltpu.SemaphoreType.DMA,
            pltpu.SemaphoreType.DMA,
        ],
        name=f"cheb_filter_sc_f{fh}x{nsplit}",
    )


_BR = 256


def _tc1_body(x_ref, w_ref, o_ref):
    o_ref[...] = jnp.dot(x_ref[...], w_ref[...],
                         preferred_element_type=jnp.float32)[None]


def _tc1(xp, W1):
    return pl.pallas_call(
        _tc1_body,
        grid=(N_PAD // _BR, 2),
        in_specs=[
            pl.BlockSpec((_BR, D_IN), lambda i, c: (i, 0)),
            pl.BlockSpec((D_IN, 128), lambda i, c: (0, c)),
        ],
        out_specs=pl.BlockSpec((1, _BR, 128), lambda i, c: (c, i, 0)),
        out_shape=jax.ShapeDtypeStruct((2, N_PAD, 128), jnp.float32),
    )(xp, W1)


def _tc2_body(coef_ref, ciso_ref, u_ref, h_ref, srec_ref, w2_ref,
              l1_ref, h2_ref):
    sr = srec_ref[:, 0:1]
    halves = []
    for c in (0, 1):
        hh = h_ref[c]
        outs = jnp.zeros_like(hh)
        for k in range(1, ORDER + 1):
            row = jnp.concatenate(
                [jnp.full((1, 64), coef_ref[2 * c, k], jnp.float32),
                 jnp.full((1, 64), coef_ref[2 * c + 1, k], jnp.float32)],
            axis=1)
            outs = outs + row * u_ref[k, c]
        c0row = jnp.concatenate(
            [jnp.full((1, 64), coef_ref[2 * c, 0], jnp.float32),
             jnp.full((1, 64), coef_ref[2 * c + 1, 0], jnp.float32)], axis=1)
        cirow = jnp.concatenate(
            [jnp.full((1, 64), ciso_ref[2 * c], jnp.float32),
             jnp.full((1, 64), ciso_ref[2 * c + 1], jnp.float32)], axis=1)
        t = jnp.where(sr > 0.0, c0row * hh + sr * outs, cirow * hh)
        halves.append(jnp.where(t > 0.0, t, jnp.exp(t) - 1.0))
    l1 = jnp.concatenate(halves, axis=1)
    l1_ref[...] = l1
    h2_ref[...] = jnp.dot(l1, w2_ref[...], preferred_element_type=jnp.float32)


def _tc2(u1, hh, srec, W2, coef1, ciso1):
    return pl.pallas_call(
        _tc2_body,
        grid=(N_PAD // _BR,),
        in_specs=[
            pl.BlockSpec(memory_space=pltpu.SMEM),
            pl.BlockSpec(memory_space=pltpu.SMEM),
            pl.BlockSpec((ORDER + 1, 2, _BR, 128), lambda i: (0, 0, i, 0)),
            pl.BlockSpec((2, _BR, 128), lambda i: (0, i, 0)),
            pl.BlockSpec((_BR, 1), lambda i: (i, 0)),
            pl.BlockSpec((HEADS * HIDDEN, N_CLASSES), lambda i: (0, 0)),
        ],
        out_specs=[
            pl.BlockSpec((_BR, HEADS * HIDDEN), lambda i: (i, 0)),
            pl.BlockSpec((_BR, N_CLASSES), lambda i: (i, 0)),
        ],
        out_shape=[
            jax.ShapeDtypeStruct((N_PAD, HEADS * HIDDEN), jnp.float32),
            jax.ShapeDtypeStruct((N_PAD, N_CLASSES), jnp.float32),
        ],
    )(coef1, ciso1, u1, hh, srec, W2)


def _tc3_body(coef_ref, ciso_ref, u_ref, h2_ref, srec_ref, lp_ref, l2_ref):
    sr = srec_ref[:, 0:1]
    h2 = h2_ref[...]
    outs = jnp.zeros_like(h2)
    for k in range(1, ORDER + 1):
        outs = outs + coef_ref[0, k] * u_ref[k, 0, :, :N_CLASSES]
    l2 = jnp.where(sr > 0.0, coef_ref[0, 0] * h2 + sr * outs,
                   ciso_ref[0] * h2)
    m = jnp.max(l2, axis=1, keepdims=True)
    sh = l2 - m
    lse = jnp.log(jnp.sum(jnp.exp(sh), axis=1, keepdims=True))
    l2_ref[...] = l2
    lp_ref[...] = sh - lse


def _tc3(u2, h2, srec, coef2, ciso2):
    return pl.pallas_call(
        _tc3_body,
        grid=(N_PAD // _BR,),
        in_specs=[
            pl.BlockSpec(memory_space=pltpu.SMEM),
            pl.BlockSpec(memory_space=pltpu.SMEM),
            pl.BlockSpec((ORDER + 1, 1, _BR, 128), lambda i: (0, 0, i, 0)),
            pl.BlockSpec((_BR, 16), lambda i: (i, 0)),
            pl.BlockSpec((_BR, 1), lambda i: (i, 0)),
        ],
        out_specs=[
            pl.BlockSpec((_BR, 16), lambda i: (i, 0)),
            pl.BlockSpec((_BR, 16), lambda i: (i, 0)),
        ],
        out_shape=[
            jax.ShapeDtypeStruct((N_PAD, N_CLASSES), jnp.float32),
            jax.ShapeDtypeStruct((N_PAD, N_CLASSES), jnp.float32),
        ],
    )(coef2, ciso2, u2, h2, srec)


def kernel(x, edge_index, W1, coef1, W2, coef2):
    src = edge_index[0].astype(jnp.int32)
    dst = edge_index[1].astype(jnp.int32)
    pad_e = EPT_PAD - EPT
    s4 = jnp.pad(src.reshape(NT, EPT), ((0, 0), (0, pad_e)),
                 constant_values=PAD_ROW).reshape(NT, NBLK, BCH, CHUNK)
    d4 = jnp.pad(dst.reshape(NT, EPT), ((0, 0), (0, pad_e)),
                 constant_values=PAD_ROW).reshape(NT, NBLK, BCH, CHUNK)
    xp = jnp.pad(x, ((0, N_PAD - N), (0, 0)))

    alt = ((-1.0) ** jnp.arange(ORDER // 2 + 1)).astype(jnp.float32)
    ciso1 = (coef1[:, ::2] * alt[None, :]).sum(axis=1)
    ciso2 = (coef2[:, ::2] * alt[None, :]).sum(axis=1)

    hh = _tc1(xp, W1)
    u1, srec = _make_filter(2, 128)(hh, s4, d4)
    srec2 = srec[:, None]
    l1p, h2p = _tc2(u1, hh, srec2, W2, coef1, ciso1)
    h2w = jnp.pad(h2p, ((0, 0), (0, 128 - N_CLASSES)))
    u2, _srec_b = _make_filter(1, 128)(h2w[None], s4, d4)
    lpp, l2p = _tc3(u2, h2p, srec2, coef2, ciso2)
    return lpp[:N], l2p[:N], l1p[:N]


# speedup vs baseline: 1.3712x; 1.0026x over previous
import jax
import jax.numpy as jnp
from jax import lax
from jax.experimental import pallas as pl
from jax.experimental.pallas import tpu as pltpu
from jax.experimental.pallas import tpu_sc as plsc

N = 10000
E = 320000
D_IN = 128
HIDDEN = 64
HEADS = 4
N_CLASSES = 16
ORDER = 16

NT = 16
N_PAD = 10240
RPT = N_PAD // NT
EPT = E // NT
CHUNK = 128
BCH = 16
NBLK = 10
EPT_PAD = NBLK * BCH * CHUNK
PAD_ROW = N_PAD - 1
NRC = 128
NROWCH = RPT // NRC


def _rsqrt_sc(v):
    vi = plsc.bitcast(v, jnp.int32)
    yi = jnp.full((16,), 0x5F3759DF, jnp.int32) - (vi >> 1)
    y = plsc.bitcast(yi, jnp.float32)
    for _ in range(4):
        y = y * (1.5 - 0.5 * v * y * y)
    return y


def _rep16(ref, r):
    return plsc.load_gather(ref, [jnp.full((16,), r, jnp.int32)])


def _make_filter(nsplit, fh, srec_in=False):
    mesh = plsc.VectorSubcoreMesh(core_axis_name="c", subcore_axis_name="s")
    nq = fh // 16

    def body(h, srcs, dsts, *rest):
        if srec_in:
            (srp, u_all, srec, sb0, sb1, db0, db1, g0,
             bones, bzd, bdeg, bdis, bdis2, sh_acc, sh_deg,
             semi0, semi1, semg0, semg1, sems0, sems1) = rest
        else:
            srp = None
            (u_all, srec, sb0, sb1, db0, db1, g0,
             bones, bzd, bdeg, bdis, bdis2, sh_acc, sh_deg,
             semi0, semi1, semg0, semg1, sems0, sems1) = rest
        c = lax.axis_index("c")
        s = lax.axis_index("s")
        base = s * RPT

        def gated(fn):
            if nsplit == 1:
                pl.when(c == 0)(fn)
            else:
                fn()
        co = c if nsplit == 2 else 0

        def _p0():
            def wz(r, _):
                for q in range(nq):
                    g0[r, pl.ds(16 * q, 16)] = jnp.zeros((16,), jnp.float32)
                return 0
            lax.fori_loop(0, 2 * NRC, wz, 0)

            def wo(m, _):
                bones[pl.ds(16 * m, 16)] = jnp.ones((16,), jnp.float32)
                return 0
            lax.fori_loop(0, CHUNK // 16, wo, 0)

            def wd(m, _):
                bzd[pl.ds(16 * m, 16)] = jnp.zeros((16,), jnp.float32)
                return 0
            lax.fori_loop(0, RPT // 16, wd, 0)

            def zch(j, _):
                pltpu.sync_copy(g0.at[pl.ds(0, NRC)],
                                sh_acc.at[pl.ds(base + j * NRC, NRC)])
                return 0
            lax.fori_loop(0, NROWCH, zch, 0)
            if not srec_in:
                pltpu.sync_copy(bzd, sh_deg.at[pl.ds(base, RPT)])
        gated(_p0)
        plsc.subcore_barrier()

        if not srec_in:
            def _p1():
                def dblk(b, _):
                    pltpu.sync_copy(dsts.at[s, b], db0)
                    for j in range(BCH):
                        pltpu.sync_copy(bones, sh_deg.at[db0.at[j]], add=True)
                    return 0
                lax.fori_loop(0, NBLK, dblk, 0)
            gated(_p1)
            plsc.subcore_barrier()

        def _p2():
            if srec_in:
                pltpu.sync_copy(srp.at[pl.ds(base, RPT)], bdeg)
            else:
                pltpu.sync_copy(sh_deg.at[pl.ds(base, RPT)], bdeg)

            def prow(m, _):
                dv = bdeg[pl.ds(16 * m, 16)]
                pos = dv > 0.0
                if srec_in:
                    y = jnp.where(pos, 1.0 / jnp.maximum(dv, 1.0), 0.0)
                    d2 = y * y
                else:
                    y = jnp.where(pos, _rsqrt_sc(jnp.maximum(dv, 1.0)), 0.0)
                    d2 = jnp.where(pos, 1.0 / jnp.maximum(dv, 1.0), 0.0)
                    bdeg[pl.ds(16 * m, 16)] = dv * y
                bdis[pl.ds(16 * m, 16)] = y
                bdis2[pl.ds(16 * m, 16)] = d2
                return 0
            lax.fori_loop(0, RPT // 16, prow, 0)

            @pl.when(c == 0)
            def _():
                pltpu.sync_copy(bdeg, srec.at[pl.ds(base, RPT)])
        gated(_p2)

        def _p3():
            def uch(j, _):
                r0 = base + j * NRC
                pltpu.sync_copy(h.at[co, pl.ds(r0, NRC)], g0.at[pl.ds(0, NRC)])

                def urow(r, _):
                    dv = _rep16(bdis, j * NRC + r)
                    for q in range(nq):
                        g0[r, pl.ds(16 * q, 16)] = dv * g0[r, pl.ds(16 * q, 16)]
                    return 0
                lax.fori_loop(0, NRC, urow, 0)
                pltpu.sync_copy(g0.at[pl.ds(0, NRC)],
                                u_all.at[0, co, pl.ds(r0, NRC)])
                return 0
            lax.fori_loop(0, NROWCH, uch, 0)
        gated(_p3)
        plsc.subcore_barrier()

        def step(i, _):
            usrc = u_all.at[i, co]

            def _gstart(sb, j, buf, sem):
                pltpu.async_copy(usrc.at[sb.at[j]], buf, sem)

            def _gwait(sb, j, buf, sem):
                pltpu.make_async_copy(usrc.at[sb.at[j]], buf, sem).wait()

            def _sstart(db, j, buf, sem):
                pltpu.async_copy(buf, sh_acc.at[db.at[j]], sem, add=True)

            def _swait(db, j, buf, sem):
                pltpu.make_async_copy(buf, sh_acc.at[db.at[j]], sem).wait()

            def _blk(b, sb, db, semi, nxt):
                pltpu.make_async_copy(srcs.at[s, b], sb, semi).wait()
                pltpu.make_async_copy(dsts.at[s, b], db, semi).wait()
                if nxt is not None:
                    bn, sbn, dbn, semn = nxt
                    pltpu.async_copy(srcs.at[s, bn], sbn, semn)
                    pltpu.async_copy(dsts.at[s, bn], dbn, semn)
                bufs = (g0.at[pl.ds(0, NRC)], g0.at[pl.ds(NRC, NRC)])
                semgs = (semg0, semg1)
                semss = (sems0, sems1)
                _gstart(sb, 0, bufs[0], semg0)
                for j in range(BCH):
                    p, o = j % 2, (j + 1) % 2
                    if j + 1 < BCH:
                        if j >= 1:
                            _swait(db, j - 1, bufs[o], semss[o])
                        _gstart(sb, j + 1, bufs[o], semgs[o])
                    _gwait(sb, j, bufs[p], semgs[p])
                    _sstart(db, j, bufs[p], semss[p])
                _swait(db, BCH - 2, bufs[BCH % 2], semss[BCH % 2])
                _swait(db, BCH - 1, bufs[(BCH + 1) % 2], semss[(BCH + 1) % 2])

            def _scat():
                pltpu.async_copy(srcs.at[s, 0], sb0, semi0)
                pltpu.async_copy(dsts.at[s, 0], db0, semi0)

                def pairs(q, _):
                    b0 = 2 * q
                    _blk(b0, sb0, db0, semi0, (b0 + 1, sb1, db1, semi1))

                    @pl.when(q < NBLK // 2 - 1)
                    def _():
                        pltpu.async_copy(srcs.at[s, b0 + 2], sb0, semi0)
                        pltpu.async_copy(dsts.at[s, b0 + 2], db0, semi0)
                    _blk(b0 + 1, sb1, db1, semi1, None)
                    return 0
                lax.fori_loop(0, NBLK // 2, pairs, 0)
            gated(_scat)
            plsc.subcore_barrier()

            fa = jnp.where(i == 0, -1.0, -2.0)
            fb = jnp.where(i == 0, 0.0, -1.0)
            im1 = lax.max(i - 1, 0)

            def _node():
                def nch(j, _):
                    r0 = base + j * NRC
                    pltpu.sync_copy(sh_acc.at[pl.ds(r0, NRC)],
                                    g0.at[pl.ds(0, NRC)])
                    pltpu.sync_copy(u_all.at[im1, co, pl.ds(r0, NRC)],
                                    g0.at[pl.ds(NRC, NRC)])

                    def nrow(r, _):
                        d2 = _rep16(bdis2, j * NRC + r)
                        for q in range(nq):
                            sl = pl.ds(16 * q, 16)
                            g0[r, sl] = (fa * d2 * g0[r, sl]
                                         + fb * g0[NRC + r, sl])
                            g0[NRC + r, sl] = jnp.zeros((16,), jnp.float32)
                        return 0
                    lax.fori_loop(0, NRC, nrow, 0)
                    pltpu.sync_copy(g0.at[pl.ds(0, NRC)],
                                    u_all.at[i + 1, co, pl.ds(r0, NRC)])
                    pltpu.sync_copy(g0.at[pl.ds(NRC, NRC)],
                                    sh_acc.at[pl.ds(r0, NRC)])
                    return 0
                lax.fori_loop(0, NROWCH, nch, 0)
            gated(_node)
            plsc.subcore_barrier()
            return 0
        lax.fori_loop(0, ORDER, step, 0)

    return pl.kernel(
        body,
        out_type=(
            jax.ShapeDtypeStruct((ORDER + 1, nsplit, N_PAD, fh), jnp.float32),
            jax.ShapeDtypeStruct((N_PAD,), jnp.float32),
        ),
        mesh=mesh,
        compiler_params=pltpu.CompilerParams(needs_layout_passes=False),
        scratch_types=[
            pltpu.VMEM((BCH, CHUNK), jnp.int32),
            pltpu.VMEM((BCH, CHUNK), jnp.int32),
            pltpu.VMEM((BCH, CHUNK), jnp.int32),
            pltpu.VMEM((BCH, CHUNK), jnp.int32),
            pltpu.VMEM((2 * NRC, fh), jnp.float32),
            pltpu.VMEM((CHUNK,), jnp.float32),
            pltpu.VMEM((RPT,), jnp.float32),
            pltpu.VMEM((RPT,), jnp.float32),
            pltpu.VMEM((RPT,), jnp.float32),
            pltpu.VMEM((RPT,), jnp.float32),
            pltpu.VMEM_SHARED((N_PAD, fh), jnp.float32),
            pltpu.VMEM_SHARED((N_PAD,), jnp.float32),
            pltpu.SemaphoreType.DMA,
            pltpu.SemaphoreType.DMA,
            pltpu.SemaphoreType.DMA,
            pltpu.SemaphoreType.DMA,
            pltpu.SemaphoreType.DMA,
            pltpu.SemaphoreType.DMA,
        ],
        name=f"cheb_filter_sc_f{fh}x{nsplit}",
    )


_BR = 256


def _tc1_body(x_ref, w_ref, o_ref):
    o_ref[...] = jnp.dot(x_ref[...], w_ref[...],
                         preferred_element_type=jnp.float32)[None]


def _tc1(xp, W1):
    return pl.pallas_call(
        _tc1_body,
        grid=(N_PAD // _BR, 2),
        in_specs=[
            pl.BlockSpec((_BR, D_IN), lambda i, c: (i, 0)),
            pl.BlockSpec((D_IN, 128), lambda i, c: (0, c)),
        ],
        out_specs=pl.BlockSpec((1, _BR, 128), lambda i, c: (c, i, 0)),
        out_shape=jax.ShapeDtypeStruct((2, N_PAD, 128), jnp.float32),
    )(xp, W1)


def _tc2_body(coef_ref, ciso_ref, u_ref, h_ref, srec_ref, w2_ref,
              l1_ref, h2_ref):
    sr = srec_ref[:, 0:1]
    halves = []
    for c in (0, 1):
        hh = h_ref[c]
        outs = jnp.zeros_like(hh)
        for k in range(1, ORDER + 1):
            row = jnp.concatenate(
                [jnp.full((1, 64), coef_ref[2 * c, k], jnp.float32),
                 jnp.full((1, 64), coef_ref[2 * c + 1, k], jnp.float32)],
            axis=1)
            outs = outs + row * u_ref[k, c]
        c0row = jnp.concatenate(
            [jnp.full((1, 64), coef_ref[2 * c, 0], jnp.float32),
             jnp.full((1, 64), coef_ref[2 * c + 1, 0], jnp.float32)], axis=1)
        cirow = jnp.concatenate(
            [jnp.full((1, 64), ciso_ref[2 * c], jnp.float32),
             jnp.full((1, 64), ciso_ref[2 * c + 1], jnp.float32)], axis=1)
        t = jnp.where(sr > 0.0, c0row * hh + sr * outs, cirow * hh)
        halves.append(jnp.where(t > 0.0, t, jnp.exp(t) - 1.0))
    l1 = jnp.concatenate(halves, axis=1)
    l1_ref[...] = l1
    h2_ref[...] = jnp.dot(l1, w2_ref[...], preferred_element_type=jnp.float32)


def _tc2(u1, hh, srec, W2, coef1, ciso1):
    return pl.pallas_call(
        _tc2_body,
        grid=(N_PAD // _BR,),
        in_specs=[
            pl.BlockSpec(memory_space=pltpu.SMEM),
            pl.BlockSpec(memory_space=pltpu.SMEM),
            pl.BlockSpec((ORDER + 1, 2, _BR, 128), lambda i: (0, 0, i, 0)),
            pl.BlockSpec((2, _BR, 128), lambda i: (0, i, 0)),
            pl.BlockSpec((_BR, 1), lambda i: (i, 0)),
            pl.BlockSpec((HEADS * HIDDEN, N_CLASSES), lambda i: (0, 0)),
        ],
        out_specs=[
            pl.BlockSpec((_BR, HEADS * HIDDEN), lambda i: (i, 0)),
            pl.BlockSpec((_BR, N_CLASSES), lambda i: (i, 0)),
        ],
        out_shape=[
            jax.ShapeDtypeStruct((N_PAD, HEADS * HIDDEN), jnp.float32),
            jax.ShapeDtypeStruct((N_PAD, N_CLASSES), jnp.float32),
        ],
    )(coef1, ciso1, u1, hh, srec, W2)


def _tc3_body(coef_ref, ciso_ref, u_ref, h2_ref, srec_ref, lp_ref, l2_ref):
    sr = srec_ref[:, 0:1]
    h2 = h2_ref[...]
    outs = jnp.zeros_like(h2)
    for k in range(1, ORDER + 1):
        outs = outs + coef_ref[0, k] * u_ref[k, 0, :, :N_CLASSES]
    l2 = jnp.where(sr > 0.0, coef_ref[0, 0] * h2 + sr * outs,
                   ciso_ref[0] * h2)
    m = jnp.max(l2, axis=1, keepdims=True)
    sh = l2 - m
    lse = jnp.log(jnp.sum(jnp.exp(sh), axis=1, keepdims=True))
    l2_ref[...] = l2
    lp_ref[...] = sh - lse


def _tc3(u2, h2, srec, coef2, ciso2):
    return pl.pallas_call(
        _tc3_body,
        grid=(N_PAD // _BR,),
        in_specs=[
            pl.BlockSpec(memory_space=pltpu.SMEM),
            pl.BlockSpec(memory_space=pltpu.SMEM),
            pl.BlockSpec((ORDER + 1, 1, _BR, 128), lambda i: (0, 0, i, 0)),
            pl.BlockSpec((_BR, 16), lambda i: (i, 0)),
            pl.BlockSpec((_BR, 1), lambda i: (i, 0)),
        ],
        out_specs=[
            pl.BlockSpec((_BR, 16), lambda i: (i, 0)),
            pl.BlockSpec((_BR, 16), lambda i: (i, 0)),
        ],
        out_shape=[
            jax.ShapeDtypeStruct((N_PAD, N_CLASSES), jnp.float32),
            jax.ShapeDtypeStruct((N_PAD, N_CLASSES), jnp.float32),
        ],
    )(coef2, ciso2, u2, h2, srec)


def kernel(x, edge_index, W1, coef1, W2, coef2):
    src = edge_index[0].astype(jnp.int32)
    dst = edge_index[1].astype(jnp.int32)
    pad_e = EPT_PAD - EPT
    s4 = jnp.pad(src.reshape(NT, EPT), ((0, 0), (0, pad_e)),
                 constant_values=PAD_ROW).reshape(NT, NBLK, BCH, CHUNK)
    d4 = jnp.pad(dst.reshape(NT, EPT), ((0, 0), (0, pad_e)),
                 constant_values=PAD_ROW).reshape(NT, NBLK, BCH, CHUNK)
    xp = jnp.pad(x, ((0, N_PAD - N), (0, 0)))

    alt = ((-1.0) ** jnp.arange(ORDER // 2 + 1)).astype(jnp.float32)
    ciso1 = (coef1[:, ::2] * alt[None, :]).sum(axis=1)
    ciso2 = (coef2[:, ::2] * alt[None, :]).sum(axis=1)

    hh = _tc1(xp, W1)
    u1, srec = _make_filter(2, 128)(hh, s4, d4)
    srec2 = srec[:, None]
    l1p, h2p = _tc2(u1, hh, srec2, W2, coef1, ciso1)
    h2w = jnp.pad(h2p, ((0, 0), (0, 128 - N_CLASSES)))
    u2, _srec_b = _make_filter(1, 128, srec_in=True)(h2w[None], s4, d4, srec)
    lpp, l2p = _tc3(u2, h2p, srec2, coef2, ciso2)
    return lpp[:N], l2p[:N], l1p[:N]
